# baseline (speedup 1.0000x reference)
import functools

import jax
import jax.numpy as jnp
from jax import lax
from jax.experimental import pallas as pl
from jax.experimental.pallas import tpu as pltpu
from jax.experimental.pallas import tpu_sc as plsc

_B = 16384
_V = 100000
_E = 128
_NC = 2
_NS = 16
_NW = _NC * _NS
_BPW = _B // _NW
_CH = 128
_NCH = _BPW // _CH
_LANES = 16


def _sc_body(ti, tj, pi, pj, bi, bj, part_o, bio, bjo,
             idx_i, idx_j, rows_i, rows_j, part_v, bib, bjb,
             sem_i, sem_j, sem_bi, sem_bj):
    wid = lax.axis_index("s") * _NC + lax.axis_index("c")
    base = wid * _BPW
    pltpu.sync_copy(pi.at[pl.ds(base, _BPW)], idx_i)
    pltpu.sync_copy(pj.at[pl.ds(base, _BPW)], idx_j)

    bias_copies = []
    for c in range(_NCH):
        sl = pl.ds(c * _CH, _CH)
        bias_copies.append(pltpu.async_copy(bi.at[idx_i.at[sl]], bib.at[sl], sem_bi))
        bias_copies.append(pltpu.async_copy(bj.at[idx_j.at[sl]], bjb.at[sl], sem_bj))

    for c in range(_NCH):
        sl = pl.ds(c * _CH, _CH)
        cp_i = pltpu.async_copy(ti.at[idx_i.at[sl]], rows_i, sem_i)
        cp_j = pltpu.async_copy(tj.at[idx_j.at[sl]], rows_j, sem_j)
        cp_i.wait()
        cp_j.wait()

        @pl.loop(0, _CH)
        def _row(r):
            acc = rows_i[r, pl.ds(0, _LANES)] * rows_j[r, pl.ds(0, _LANES)]
            for k in range(1, _E // _LANES):
                acc = acc + (rows_i[r, pl.ds(k * _LANES, _LANES)]
                             * rows_j[r, pl.ds(k * _LANES, _LANES)])
            part_v[r, :] = acc

        pltpu.sync_copy(part_v, part_o.at[pl.ds(base + c * _CH, _CH)])

    for cp in bias_copies:
        cp.wait()
    pltpu.sync_copy(bib, bio.at[pl.ds(base, _BPW)])
    pltpu.sync_copy(bjb, bjo.at[pl.ds(base, _BPW)])


@functools.lru_cache(maxsize=1)
def _sc_gather_dot():
    return pl.kernel(
        _sc_body,
        mesh=plsc.VectorSubcoreMesh(core_axis_name="c", subcore_axis_name="s"),
        out_type=(
            jax.ShapeDtypeStruct((_B, _LANES), jnp.float32),
            jax.ShapeDtypeStruct((_B,), jnp.float32),
            jax.ShapeDtypeStruct((_B,), jnp.float32),
        ),
        scratch_types=[
            pltpu.VMEM((_BPW,), jnp.int32),
            pltpu.VMEM((_BPW,), jnp.int32),
            pltpu.VMEM((_CH, _E), jnp.float32),
            pltpu.VMEM((_CH, _E), jnp.float32),
            pltpu.VMEM((_CH, _LANES), jnp.float32),
            pltpu.VMEM((_BPW,), jnp.float32),
            pltpu.VMEM((_BPW,), jnp.float32),
            pltpu.SemaphoreType.DMA,
            pltpu.SemaphoreType.DMA,
            pltpu.SemaphoreType.DMA,
            pltpu.SemaphoreType.DMA,
        ],
    )


def _tc_body(part_ref, bi_ref, bj_ref, wij_ref, wf_ref, out_ref):
    score = jnp.sum(part_ref[...], axis=1, keepdims=True)
    d = score + bi_ref[...] + bj_ref[...] - jnp.log(wij_ref[...])
    val = jnp.sum(d * d * wf_ref[...], keepdims=True)

    @pl.when(pl.program_id(0) == 0)
    def _():
        out_ref[...] = jnp.zeros_like(out_ref)

    out_ref[...] += val


def _loss_tc(part, big, bjg, wij2, wf2):
    grid = 8
    blk = _B // grid
    return pl.pallas_call(
        _tc_body,
        grid=(grid,),
        in_specs=[
            pl.BlockSpec((blk, _LANES), lambda i: (i, 0)),
            pl.BlockSpec((blk, 1), lambda i: (i, 0)),
            pl.BlockSpec((blk, 1), lambda i: (i, 0)),
            pl.BlockSpec((blk, 1), lambda i: (i, 0)),
            pl.BlockSpec((blk, 1), lambda i: (i, 0)),
        ],
        out_specs=pl.BlockSpec((1, 1), lambda i: (0, 0)),
        out_shape=jax.ShapeDtypeStruct((1, 1), jnp.float32),
    )(part, big, bjg, wij2, wf2)


def kernel(pos_i, pos_j, wij, wf, input_embs, output_embs, bi_table, bj_table):
    pi = pos_i.reshape(_B).astype(jnp.int32)
    pj = pos_j.reshape(_B).astype(jnp.int32)
    part, big, bjg = _sc_gather_dot()(
        input_embs, output_embs, pi, pj,
        bi_table.reshape(_V), bj_table.reshape(_V))
    out = _loss_tc(part,
                   big.reshape(_B, 1), bjg.reshape(_B, 1),
                   wij.reshape(_B, 1), wf.reshape(_B, 1))
    return out.reshape(())


# speedup vs baseline: 1.7757x; 1.7757x over previous
import dataclasses
import functools

import jax
import jax.numpy as jnp
from jax import lax
from jax.experimental import pallas as pl
from jax.experimental.pallas import tpu as pltpu
from jax.experimental.pallas import tpu_sc as plsc

_B = 16384
_V = 100000
_E = 128
_NC = 2
_NS = 16
_NW = _NC * _NS
_BPW = _B // _NW
_CH = 128
_NCH = _BPW // _CH
_L = 16


def _sc_body(ti, tj, pi, pj, bi, bj, score_o, bio, bjo,
             idx_i, idx_j, ri0, rj0, ri1, rj1, part_v, score_v, bib, bjb,
             sem_i0, sem_j0, sem_i1, sem_j1, sem_bi, sem_bj):
    wid = lax.axis_index("s") * _NC + lax.axis_index("c")
    base = wid * _BPW
    pltpu.sync_copy(pi.at[pl.ds(base, _BPW)], idx_i)
    pltpu.sync_copy(pj.at[pl.ds(base, _BPW)], idx_j)

    bias_copies = []
    for c in range(_NCH):
        sl = pl.ds(c * _CH, _CH)
        bias_copies.append(pltpu.async_copy(bi.at[idx_i.at[sl]], bib.at[sl], sem_bi))
        bias_copies.append(pltpu.async_copy(bj.at[idx_j.at[sl]], bjb.at[sl], sem_bj))

    rbufs = ((ri0, rj0, sem_i0, sem_j0), (ri1, rj1, sem_i1, sem_j1))

    def issue(c):
        sl = pl.ds(c * _CH, _CH)
        ri, rj, si, sj = rbufs[c % 2]
        return (pltpu.async_copy(ti.at[idx_i.at[sl]], ri, si),
                pltpu.async_copy(tj.at[idx_j.at[sl]], rj, sj))

    lane = lax.iota(jnp.int32, _L)

    inflight = issue(0)
    for c in range(_NCH):
        cp_i, cp_j = inflight
        if c + 1 < _NCH:
            nxt = issue(c + 1)
        cp_i.wait()
        cp_j.wait()
        ri, rj, _, _ = rbufs[c % 2]

        @pl.loop(0, _CH)
        def _row(r):
            acc = ri[r, pl.ds(0, _L)] * rj[r, pl.ds(0, _L)]
            for k in range(1, _E // _L):
                acc = acc + (ri[r, pl.ds(k * _L, _L)]
                             * rj[r, pl.ds(k * _L, _L)])
            part_v[pl.ds(r * _L, _L)] = acc

        @pl.loop(0, _CH // _L)
        def _grp(g):
            gbase = g * (_L * _L) + lane * _L
            acc = plsc.load_gather(part_v, [gbase])
            for k in range(1, _L):
                acc = acc + plsc.load_gather(part_v, [gbase + k])
            score_v[pl.ds(c * _CH + g * _L, _L)] = acc

        if c + 1 < _NCH:
            inflight = nxt

    pltpu.sync_copy(score_v, score_o.at[pl.ds(base, _BPW)])
    for cp in bias_copies:
        cp.wait()
    pltpu.sync_copy(bib, bio.at[pl.ds(base, _BPW)])
    pltpu.sync_copy(bjb, bjo.at[pl.ds(base, _BPW)])


@functools.lru_cache(maxsize=1)
def _sc_gather_dot():
    cp = pltpu.CompilerParams()
    if "needs_layout_passes" in pltpu.CompilerParams.__dataclass_fields__:
        cp = dataclasses.replace(cp, needs_layout_passes=False)
    return pl.kernel(
        _sc_body,
        mesh=plsc.VectorSubcoreMesh(core_axis_name="c", subcore_axis_name="s"),
        compiler_params=cp,
        out_type=(
            jax.ShapeDtypeStruct((_B,), jnp.float32),
            jax.ShapeDtypeStruct((_B,), jnp.float32),
            jax.ShapeDtypeStruct((_B,), jnp.float32),
        ),
        scratch_types=[
            pltpu.VMEM((_BPW,), jnp.int32),
            pltpu.VMEM((_BPW,), jnp.int32),
            pltpu.VMEM((_CH, _E), jnp.float32),
            pltpu.VMEM((_CH, _E), jnp.float32),
            pltpu.VMEM((_CH, _E), jnp.float32),
            pltpu.VMEM((_CH, _E), jnp.float32),
            pltpu.VMEM((_CH * _L,), jnp.float32),
            pltpu.VMEM((_BPW,), jnp.float32),
            pltpu.VMEM((_BPW,), jnp.float32),
            pltpu.VMEM((_BPW,), jnp.float32),
            pltpu.SemaphoreType.DMA,
            pltpu.SemaphoreType.DMA,
            pltpu.SemaphoreType.DMA,
            pltpu.SemaphoreType.DMA,
            pltpu.SemaphoreType.DMA,
            pltpu.SemaphoreType.DMA,
        ],
    )


def _tc_body(score_ref, bi_ref, bj_ref, wij_ref, wf_ref, out_ref):
    d = score_ref[...] + bi_ref[...] + bj_ref[...] - jnp.log(wij_ref[...])
    out_ref[...] = jnp.sum(d * d * wf_ref[...], keepdims=True)


def _loss_tc(score, big, bjg, wij2, wf2):
    return pl.pallas_call(
        _tc_body,
        out_shape=jax.ShapeDtypeStruct((1, 1), jnp.float32),
    )(score, big, bjg, wij2, wf2)


def kernel(pos_i, pos_j, wij, wf, input_embs, output_embs, bi_table, bj_table):
    pi = pos_i.reshape(_B).astype(jnp.int32)
    pj = pos_j.reshape(_B).astype(jnp.int32)
    score, big, bjg = _sc_gather_dot()(
        input_embs, output_embs, pi, pj,
        bi_table.reshape(_V), bj_table.reshape(_V))
    sq = _B // 128
    out = _loss_tc(score.reshape(sq, 128), big.reshape(sq, 128),
                   bjg.reshape(sq, 128), wij.reshape(sq, 128),
                   wf.reshape(sq, 128))
    return out.reshape(())
